# baseline (speedup 1.0000x reference)
import functools

import jax
import jax.numpy as jnp
from jax import lax
from jax.experimental import pallas as pl
from jax.experimental.pallas import tpu as pltpu
from jax.experimental.pallas import tpu_sc as plsc

B, T, DIM = 1, 2048, 1024
HID = 2048
E = 8
TM = 128
NP = 5120
G = NP // TM
NSUB = 32
TPW = T // NSUB
L = 16
SW = 128


def _gather16(x, idx):
    dnums = lax.GatherDimensionNumbers(
        offset_dims=(), collapsed_slice_dims=(0,), start_index_map=(0,))
    return lax.gather(x, idx[:, None], dnums, (1,),
                      mode=lax.GatherScatterMode.PROMISE_IN_BOUNDS)


def _cumsum16(x):
    lane = lax.iota(jnp.int32, L)
    for k in (1, 2, 4, 8):
        sh = _gather16(x, jnp.maximum(lane - k, 0))
        x = x + jnp.where(lane >= k, sh, 0)
    return x


def _allsum16(x):
    lane = lax.iota(jnp.int32, L)
    for k in (8, 4, 2, 1):
        x = x + _gather16(x, lane ^ k)
    return x


def _gate_kernel(wg_ref, x_ref, s_ref):
    s_ref[...] = jax.lax.dot_general(
        wg_ref[...], x_ref[...], (((1,), (1,)), ((), ())),
        preferred_element_type=jnp.float32)


def _route_kernel(scores_hbm, x_hbm, xs_hbm, pos_hbm, scl_hbm, texp_hbm,
                  sc_v, e1_v, e2_v, p1_v, p2_v, cnt_v, cntb_v, run_v,
                  tok_v, dst_v, sclsrc_v, gtex_v, tmp_v, rows_v, sem):
    cid = lax.axis_index("c")
    sid = lax.axis_index("s")
    wid = sid * 2 + cid
    tbase = wid * TPW

    lane = lax.iota(jnp.int32, L)

    pltpu.sync_copy(scores_hbm, sc_v)
    cnt_v[...] = jnp.zeros((L,), jnp.int32)

    def scan_body(v, _):
        @pl.when(v == wid * (TPW // L))
        def _():
            cntb_v[...] = cnt_v[...]

        m1 = sc_v[0, pl.ds(v * L, L)]
        i1 = jnp.zeros((L,), jnp.int32)
        m2 = jnp.full((L,), -1e30, jnp.float32)
        i2 = jnp.zeros((L,), jnp.int32)
        for e in range(1, E):
            se = sc_v[e, pl.ds(v * L, L)]
            gt1 = se > m1
            gt2 = se > m2
            nm2 = jnp.where(gt1, m1, jnp.where(gt2, se, m2))
            ni2 = jnp.where(gt1, i1, jnp.where(gt2, e, i2))
            m1 = jnp.where(gt1, se, m1)
            i1 = jnp.where(gt1, e, i1)
            m2, i2 = nm2, ni2
        p1 = 1.0 / (1.0 + jnp.exp(m2 - m1))
        e1_v[pl.ds(v * L, L)] = i1
        e2_v[pl.ds(v * L, L)] = i2
        p1_v[pl.ds(v * L, L)] = p1
        p2_v[pl.ds(v * L, L)] = 1.0 - p1
        add = jnp.zeros((L,), jnp.int32)
        for e in range(E):
            pc = _allsum16(jnp.where(i1 == e, 1, 0)
                           + jnp.where(i2 == e, 1, 0))
            add = add + jnp.where(lane == e, pc, 0)
        cnt_v[...] = cnt_v[...] + add
        return 0

    lax.fori_loop(0, T // L, scan_body, 0)

    tot = cnt_v[...]
    padded = (tot + (TM - 1)) & (~(TM - 1))
    cume = _cumsum16(padded)
    base = cume - padded
    run_v[...] = base + cntb_v[...]

    @pl.when(wid == 0)
    def _():
        segcarry = jnp.zeros((L,), jnp.int32)
        prev_last = jnp.zeros((L,), jnp.int32)
        for j in range(3):
            start = (lane + j * L) * TM
            te = jnp.zeros((L,), jnp.int32)
            for e in range(E):
                ce = _gather16(cume, jnp.full((L,), e, jnp.int32))
                te = te + jnp.where(start >= ce, 1, 0)
            te = jnp.minimum(te, E - 1)
            prevv = _gather16(te, jnp.maximum(lane - 1, 0))
            prevv = jnp.where(lane == 0,
                              prev_last if j > 0 else te, prevv)
            d = jnp.where(te != prevv, 1, 0)
            segsv = _cumsum16(d) + segcarry
            segcarry = _gather16(segsv, jnp.full((L,), L - 1, jnp.int32))
            prev_last = _gather16(te, jnp.full((L,), L - 1, jnp.int32))
            gtex_v[pl.ds(j * L, L)] = te
            gtex_v[pl.ds(48 + j * L, L)] = segsv
        pltpu.sync_copy(gtex_v, texp_hbm)

    for k in range(2):
        ek_v = e1_v if k == 0 else e2_v
        pk_v = p1_v if k == 0 else p2_v
        for vv in range(TPW // L):
            ev = ek_v[pl.ds(tbase + vv * L, L)]
            dest = jnp.zeros((L,), jnp.int32)
            run = run_v[...]
            upd = jnp.zeros((L,), jnp.int32)
            for e in range(E):
                m = ev == e
                r = _cumsum16(jnp.where(m, 1, 0))
                bs = _gather16(run, jnp.full((L,), e, jnp.int32))
                dest = jnp.where(m, bs + r - 1, dest)
                pc = _gather16(r, jnp.full((L,), L - 1, jnp.int32))
                upd = upd + jnp.where(lane == e, pc, 0)
            run_v[...] = run + upd
            dst_v[pl.ds(vv * L, L)] = dest
            tok_v[pl.ds(vv * L, L)] = tbase + vv * L + lane
        pltpu.sync_copy(dst_v, pos_hbm.at[k, pl.ds(tbase, TPW)])

        def scl_body(j, _):
            pv = pk_v[pl.ds(tbase + (j & ~(L - 1)), L)]
            sp = _gather16(pv, jnp.full((L,), 1, jnp.int32) * (j & (L - 1)))
            tmp_v[...] = sp
            spn = tmp_v[...]
            for q in range(SW // L):
                sclsrc_v[j, pl.ds(q * L, L)] = spn
            return 0

        lax.fori_loop(0, TPW, scl_body, 0)

        pltpu.async_copy(x_hbm.at[tok_v], rows_v, sem).wait()
        pltpu.async_copy(rows_v, xs_hbm.at[dst_v], sem).wait()
        pltpu.async_copy(sclsrc_v, scl_hbm.at[dst_v], sem).wait()


def _route(scoresT, xf):
    mesh = plsc.VectorSubcoreMesh(core_axis_name="c", subcore_axis_name="s")
    f = functools.partial(
        pl.kernel,
        out_type=[
            jax.ShapeDtypeStruct((NP, DIM), jnp.float32),
            jax.ShapeDtypeStruct((2, T), jnp.int32),
            jax.ShapeDtypeStruct((NP, SW), jnp.float32),
            jax.ShapeDtypeStruct((96,), jnp.int32),
        ],
        mesh=mesh,
        scratch_types=[
            pltpu.VMEM((E, T), jnp.float32),
            pltpu.VMEM((T,), jnp.int32),
            pltpu.VMEM((T,), jnp.int32),
            pltpu.VMEM((T,), jnp.float32),
            pltpu.VMEM((T,), jnp.float32),
            pltpu.VMEM((L,), jnp.int32),
            pltpu.VMEM((L,), jnp.int32),
            pltpu.VMEM((L,), jnp.int32),
            pltpu.VMEM((TPW,), jnp.int32),
            pltpu.VMEM((TPW,), jnp.int32),
            pltpu.VMEM((TPW, SW), jnp.float32),
            pltpu.VMEM((96,), jnp.int32),
            pltpu.VMEM((L,), jnp.float32),
            pltpu.VMEM((TPW, DIM), jnp.float32),
            pltpu.SemaphoreType.DMA,
        ],
    )(_route_kernel)
    return f(scoresT, xf)


def _ffn_kernel(mt_ref, xs_ref, scl_ref, w1_hbm, w2_hbm, w3_hbm, ys_ref,
                w1b, w2b, w3b, sems):
    g = pl.program_id(0)
    te = mt_ref[g]
    seg = mt_ref[48 + g]
    slot = seg & 1

    def _copies(e, sl):
        return (pltpu.make_async_copy(w1_hbm.at[e], w1b.at[sl], sems.at[sl]),
                pltpu.make_async_copy(w2_hbm.at[e], w2b.at[sl], sems.at[sl]),
                pltpu.make_async_copy(w3_hbm.at[e], w3b.at[sl], sems.at[sl]))

    @pl.when(g == 0)
    def _():
        for c in _copies(te, slot):
            c.start()

    @pl.when((g == 0) | (seg != mt_ref[48 + jnp.maximum(g - 1, 0)]))
    def _():
        for c in _copies(te, slot):
            c.wait()

    xt = xs_ref[...]
    a = jax.lax.dot_general(xt, w1b[slot], (((1,), (1,)), ((), ())),
                            preferred_element_type=jnp.float32)
    b = jax.lax.dot_general(xt, w2b[slot], (((1,), (1,)), ((), ())),
                            preferred_element_type=jnp.float32)
    h = (a * jax.nn.sigmoid(a)) * b
    y = jax.lax.dot_general(h, w3b[slot], (((1,), (1,)), ((), ())),
                            preferred_element_type=jnp.float32)
    ys_ref[...] = y * scl_ref[:, 0:1]

    @pl.when((g < G - 1) & (mt_ref[jnp.minimum(g + 1, G - 1)] != te))
    def _():
        for c in _copies(mt_ref[jnp.minimum(g + 1, G - 1)], 1 - slot):
            c.start()


def _ffn(tmaps, xs, scl, W1, W2, W3):
    grid_spec = pltpu.PrefetchScalarGridSpec(
        num_scalar_prefetch=1,
        grid=(G,),
        in_specs=[
            pl.BlockSpec((TM, DIM), lambda g, mt: (g, 0)),
            pl.BlockSpec((TM, SW), lambda g, mt: (g, 0)),
            pl.BlockSpec(memory_space=pltpu.MemorySpace.HBM),
            pl.BlockSpec(memory_space=pltpu.MemorySpace.HBM),
            pl.BlockSpec(memory_space=pltpu.MemorySpace.HBM),
        ],
        out_specs=pl.BlockSpec((TM, DIM), lambda g, mt: (g, 0)),
        scratch_shapes=[
            pltpu.MemorySpace.VMEM((2, HID, DIM), jnp.float32),
            pltpu.MemorySpace.VMEM((2, HID, DIM), jnp.float32),
            pltpu.MemorySpace.VMEM((2, DIM, HID), jnp.float32),
            pltpu.SemaphoreType.DMA((2,)),
        ],
    )
    return pl.pallas_call(
        _ffn_kernel,
        grid_spec=grid_spec,
        out_shape=jax.ShapeDtypeStruct((NP, DIM), jnp.float32),
        compiler_params=pltpu.CompilerParams(
            dimension_semantics=("arbitrary",),
        ),
    )(tmaps, xs, scl, W1, W2, W3)


CH = 32


def _combine_kernel(ys_hbm, pos_hbm, out_hbm,
                    pos0_v, pos1_v, r0_v, r1_v, o_v, sem):
    cid = lax.axis_index("c")
    sid = lax.axis_index("s")
    wid = sid * 2 + cid
    tbase = wid * TPW

    for c in range(TPW // CH):
        cb = tbase + c * CH
        pltpu.sync_copy(pos_hbm.at[0, pl.ds(cb, CH)], pos0_v)
        pltpu.sync_copy(pos_hbm.at[1, pl.ds(cb, CH)], pos1_v)
        pltpu.async_copy(ys_hbm.at[pos0_v], r0_v, sem).wait()
        pltpu.async_copy(ys_hbm.at[pos1_v], r1_v, sem).wait()

        def tok_body(t, _):
            for d in range(DIM // L):
                o_v[t, pl.ds(d * L, L)] = (r0_v[t, pl.ds(d * L, L)]
                                           + r1_v[t, pl.ds(d * L, L)])
            return 0

        lax.fori_loop(0, CH, tok_body, 0)
        pltpu.sync_copy(o_v, out_hbm.at[pl.ds(cb, CH)])


def _combine(ys, pos2):
    mesh = plsc.VectorSubcoreMesh(core_axis_name="c", subcore_axis_name="s")
    f = functools.partial(
        pl.kernel,
        out_type=[jax.ShapeDtypeStruct((T, DIM), jnp.float32)],
        mesh=mesh,
        scratch_types=[
            pltpu.VMEM((CH,), jnp.int32),
            pltpu.VMEM((CH,), jnp.int32),
            pltpu.VMEM((CH, DIM), jnp.float32),
            pltpu.VMEM((CH, DIM), jnp.float32),
            pltpu.VMEM((CH, DIM), jnp.float32),
            pltpu.SemaphoreType.DMA,
        ],
    )(_combine_kernel)
    return f(ys, pos2)[0]


def kernel(x, Wg, W1, W2, W3):
    xf = x.reshape(T, DIM)
    scoresT = pl.pallas_call(
        _gate_kernel,
        out_shape=jax.ShapeDtypeStruct((E, T), jnp.float32),
    )(Wg, xf)
    xs, pos2, scl, tmaps = _route(scoresT, xf)
    ys = _ffn(tmaps, xs, scl, W1, W2, W3)
    out = _combine(ys, pos2)
    return out.reshape(B, T, DIM)


# speedup vs baseline: 1.0936x; 1.0936x over previous
import functools

import jax
import jax.numpy as jnp
from jax import lax
from jax.experimental import pallas as pl
from jax.experimental.pallas import tpu as pltpu
from jax.experimental.pallas import tpu_sc as plsc

B, T, DIM = 1, 2048, 1024
HID = 2048
E = 8
TM = 128
NP = 5120
G = NP // TM
NSUB = 32
TPW = T // NSUB
L = 16
SW = 128


def _gather16(x, idx):
    dnums = lax.GatherDimensionNumbers(
        offset_dims=(), collapsed_slice_dims=(0,), start_index_map=(0,))
    return lax.gather(x, idx[:, None], dnums, (1,),
                      mode=lax.GatherScatterMode.PROMISE_IN_BOUNDS)


def _cumsum16(x):
    lane = lax.iota(jnp.int32, L)
    for k in (1, 2, 4, 8):
        sh = _gather16(x, jnp.maximum(lane - k, 0))
        x = x + jnp.where(lane >= k, sh, 0)
    return x


def _allsum16(x):
    lane = lax.iota(jnp.int32, L)
    for k in (8, 4, 2, 1):
        x = x + _gather16(x, lane ^ k)
    return x


def _gate_kernel(wg_ref, x_ref, s_ref):
    s_ref[...] = jax.lax.dot_general(
        wg_ref[...], x_ref[...], (((1,), (1,)), ((), ())),
        preferred_element_type=jnp.float32)


def _route_kernel(scores_hbm, x_hbm, xs_hbm, pos_hbm, scl_hbm, texp_hbm,
                  sc_v, e1_v, e2_v, p1_v, p2_v, cnt_v, cntb_v, run_v,
                  tok_v, dst_v, sclsrc_v, gtex_v, tmp_v, rows_v, sem):
    cid = lax.axis_index("c")
    sid = lax.axis_index("s")
    wid = sid * 2 + cid
    tbase = wid * TPW

    lane = lax.iota(jnp.int32, L)

    pltpu.sync_copy(scores_hbm, sc_v)
    cnt_v[...] = jnp.zeros((L,), jnp.int32)

    def scan_body(v, _):
        @pl.when(v == wid * (TPW // L))
        def _():
            cntb_v[...] = cnt_v[...]

        m1 = sc_v[0, pl.ds(v * L, L)]
        i1 = jnp.zeros((L,), jnp.int32)
        m2 = jnp.full((L,), -1e30, jnp.float32)
        i2 = jnp.zeros((L,), jnp.int32)
        for e in range(1, E):
            se = sc_v[e, pl.ds(v * L, L)]
            gt1 = se > m1
            gt2 = se > m2
            nm2 = jnp.where(gt1, m1, jnp.where(gt2, se, m2))
            ni2 = jnp.where(gt1, i1, jnp.where(gt2, e, i2))
            m1 = jnp.where(gt1, se, m1)
            i1 = jnp.where(gt1, e, i1)
            m2, i2 = nm2, ni2
        p1 = 1.0 / (1.0 + jnp.exp(m2 - m1))
        e1_v[pl.ds(v * L, L)] = i1
        e2_v[pl.ds(v * L, L)] = i2
        p1_v[pl.ds(v * L, L)] = p1
        p2_v[pl.ds(v * L, L)] = 1.0 - p1
        add = jnp.zeros((L,), jnp.int32)
        for e in range(E):
            pc = _allsum16(jnp.where(i1 == e, 1, 0)
                           + jnp.where(i2 == e, 1, 0))
            add = add + jnp.where(lane == e, pc, 0)
        cnt_v[...] = cnt_v[...] + add
        return 0

    lax.fori_loop(0, T // L, scan_body, 0)

    tot = cnt_v[...]
    padded = (tot + (TM - 1)) & (~(TM - 1))
    cume = _cumsum16(padded)
    base = cume - padded
    run_v[...] = base + cntb_v[...]

    @pl.when(wid == 0)
    def _():
        segcarry = jnp.zeros((L,), jnp.int32)
        prev_last = jnp.zeros((L,), jnp.int32)
        for j in range(3):
            start = (lane + j * L) * TM
            te = jnp.zeros((L,), jnp.int32)
            for e in range(E):
                ce = _gather16(cume, jnp.full((L,), e, jnp.int32))
                te = te + jnp.where(start >= ce, 1, 0)
            te = jnp.minimum(te, E - 1)
            prevv = _gather16(te, jnp.maximum(lane - 1, 0))
            prevv = jnp.where(lane == 0,
                              prev_last if j > 0 else te, prevv)
            d = jnp.where(te != prevv, 1, 0)
            segsv = _cumsum16(d) + segcarry
            segcarry = _gather16(segsv, jnp.full((L,), L - 1, jnp.int32))
            prev_last = _gather16(te, jnp.full((L,), L - 1, jnp.int32))
            gtex_v[pl.ds(j * L, L)] = te
            gtex_v[pl.ds(48 + j * L, L)] = segsv
        pltpu.sync_copy(gtex_v, texp_hbm)

    for k in range(2):
        ek_v = e1_v if k == 0 else e2_v
        pk_v = p1_v if k == 0 else p2_v
        for vv in range(TPW // L):
            ev = ek_v[pl.ds(tbase + vv * L, L)]
            dest = jnp.zeros((L,), jnp.int32)
            run = run_v[...]
            upd = jnp.zeros((L,), jnp.int32)
            for e in range(E):
                m = ev == e
                r = _cumsum16(jnp.where(m, 1, 0))
                bs = _gather16(run, jnp.full((L,), e, jnp.int32))
                dest = jnp.where(m, bs + r - 1, dest)
                pc = _gather16(r, jnp.full((L,), L - 1, jnp.int32))
                upd = upd + jnp.where(lane == e, pc, 0)
            run_v[...] = run + upd
            dst_v[pl.ds(vv * L, L)] = dest
            tok_v[pl.ds(vv * L, L)] = tbase + vv * L + lane
        pltpu.sync_copy(dst_v, pos_hbm.at[k, pl.ds(tbase, TPW)])

        def scl_body(j, _):
            pv = pk_v[pl.ds(tbase + (j & ~(L - 1)), L)]
            sp = _gather16(pv, jnp.full((L,), 1, jnp.int32) * (j & (L - 1)))
            tmp_v[...] = sp
            spn = tmp_v[...]
            for q in range(SW // L):
                sclsrc_v[j, pl.ds(q * L, L)] = spn
            return 0

        lax.fori_loop(0, TPW, scl_body, 0)

        pltpu.async_copy(x_hbm.at[tok_v], rows_v, sem).wait()
        pltpu.async_copy(rows_v, xs_hbm.at[dst_v], sem).wait()
        pltpu.async_copy(sclsrc_v, scl_hbm.at[dst_v], sem).wait()


def _route(scoresT, xf):
    mesh = plsc.VectorSubcoreMesh(core_axis_name="c", subcore_axis_name="s")
    f = functools.partial(
        pl.kernel,
        out_type=[
            jax.ShapeDtypeStruct((NP, DIM), jnp.float32),
            jax.ShapeDtypeStruct((2, T), jnp.int32),
            jax.ShapeDtypeStruct((NP, SW), jnp.float32),
            jax.ShapeDtypeStruct((96,), jnp.int32),
        ],
        mesh=mesh,
        scratch_types=[
            pltpu.VMEM((E, T), jnp.float32),
            pltpu.VMEM((T,), jnp.int32),
            pltpu.VMEM((T,), jnp.int32),
            pltpu.VMEM((T,), jnp.float32),
            pltpu.VMEM((T,), jnp.float32),
            pltpu.VMEM((L,), jnp.int32),
            pltpu.VMEM((L,), jnp.int32),
            pltpu.VMEM((L,), jnp.int32),
            pltpu.VMEM((TPW,), jnp.int32),
            pltpu.VMEM((TPW,), jnp.int32),
            pltpu.VMEM((TPW, SW), jnp.float32),
            pltpu.VMEM((96,), jnp.int32),
            pltpu.VMEM((L,), jnp.float32),
            pltpu.VMEM((TPW, DIM), jnp.float32),
            pltpu.SemaphoreType.DMA,
        ],
    )(_route_kernel)
    return f(scoresT, xf)


def _ffn_kernel(mt_ref, xs_ref, scl_ref, w1_hbm, w2_hbm, w3_hbm, ys_ref,
                w1b, w2b, w3b, sems):
    g = pl.program_id(0)
    te = mt_ref[g]
    seg = mt_ref[48 + g]
    slot = seg & 1

    def _copies(e, sl):
        return (pltpu.make_async_copy(w1_hbm.at[e], w1b.at[sl], sems.at[sl]),
                pltpu.make_async_copy(w2_hbm.at[e], w2b.at[sl], sems.at[sl]),
                pltpu.make_async_copy(w3_hbm.at[e], w3b.at[sl], sems.at[sl]))

    @pl.when(g == 0)
    def _():
        for c in _copies(te, slot):
            c.start()

    @pl.when((g == 0) | (seg != mt_ref[48 + jnp.maximum(g - 1, 0)]))
    def _():
        for c in _copies(te, slot):
            c.wait()

    xt = xs_ref[...]
    at = jax.lax.dot_general(w1b[slot], xt, (((1,), (1,)), ((), ())),
                             preferred_element_type=jnp.float32)
    bt = jax.lax.dot_general(w2b[slot], xt, (((1,), (1,)), ((), ())),
                             preferred_element_type=jnp.float32)
    ht = (at * jax.nn.sigmoid(at)) * bt
    y = jax.lax.dot_general(ht, w3b[slot], (((0,), (1,)), ((), ())),
                            preferred_element_type=jnp.float32)
    ys_ref[...] = y * scl_ref[:, 0:1]

    @pl.when((g < G - 1) & (mt_ref[jnp.minimum(g + 1, G - 1)] != te))
    def _():
        for c in _copies(mt_ref[jnp.minimum(g + 1, G - 1)], 1 - slot):
            c.start()


def _ffn(tmaps, xs, scl, W1, W2, W3):
    grid_spec = pltpu.PrefetchScalarGridSpec(
        num_scalar_prefetch=1,
        grid=(G,),
        in_specs=[
            pl.BlockSpec((TM, DIM), lambda g, mt: (g, 0)),
            pl.BlockSpec((TM, SW), lambda g, mt: (g, 0)),
            pl.BlockSpec(memory_space=pltpu.MemorySpace.HBM),
            pl.BlockSpec(memory_space=pltpu.MemorySpace.HBM),
            pl.BlockSpec(memory_space=pltpu.MemorySpace.HBM),
        ],
        out_specs=pl.BlockSpec((TM, DIM), lambda g, mt: (g, 0)),
        scratch_shapes=[
            pltpu.MemorySpace.VMEM((2, HID, DIM), jnp.float32),
            pltpu.MemorySpace.VMEM((2, HID, DIM), jnp.float32),
            pltpu.MemorySpace.VMEM((2, DIM, HID), jnp.float32),
            pltpu.SemaphoreType.DMA((2,)),
        ],
    )
    return pl.pallas_call(
        _ffn_kernel,
        grid_spec=grid_spec,
        out_shape=jax.ShapeDtypeStruct((NP, DIM), jnp.float32),
        compiler_params=pltpu.CompilerParams(
            dimension_semantics=("arbitrary",),
        ),
    )(tmaps, xs, scl, W1, W2, W3)


CH = 32


def _combine_kernel(ys_hbm, pos_hbm, out_hbm,
                    pos0_v, pos1_v, r0_v, r1_v, o_v, sem):
    cid = lax.axis_index("c")
    sid = lax.axis_index("s")
    wid = sid * 2 + cid
    tbase = wid * TPW

    for c in range(TPW // CH):
        cb = tbase + c * CH
        pltpu.sync_copy(pos_hbm.at[0, pl.ds(cb, CH)], pos0_v)
        pltpu.sync_copy(pos_hbm.at[1, pl.ds(cb, CH)], pos1_v)
        pltpu.async_copy(ys_hbm.at[pos0_v], r0_v, sem).wait()
        pltpu.async_copy(ys_hbm.at[pos1_v], r1_v, sem).wait()

        def tok_body(t, _):
            for d in range(DIM // L):
                o_v[t, pl.ds(d * L, L)] = (r0_v[t, pl.ds(d * L, L)]
                                           + r1_v[t, pl.ds(d * L, L)])
            return 0

        lax.fori_loop(0, CH, tok_body, 0)
        pltpu.sync_copy(o_v, out_hbm.at[pl.ds(cb, CH)])


def _combine(ys, pos2):
    mesh = plsc.VectorSubcoreMesh(core_axis_name="c", subcore_axis_name="s")
    f = functools.partial(
        pl.kernel,
        out_type=[jax.ShapeDtypeStruct((T, DIM), jnp.float32)],
        mesh=mesh,
        scratch_types=[
            pltpu.VMEM((CH,), jnp.int32),
            pltpu.VMEM((CH,), jnp.int32),
            pltpu.VMEM((CH, DIM), jnp.float32),
            pltpu.VMEM((CH, DIM), jnp.float32),
            pltpu.VMEM((CH, DIM), jnp.float32),
            pltpu.SemaphoreType.DMA,
        ],
    )(_combine_kernel)
    return f(ys, pos2)[0]


def kernel(x, Wg, W1, W2, W3):
    xf = x.reshape(T, DIM)
    scoresT = pl.pallas_call(
        _gate_kernel,
        out_shape=jax.ShapeDtypeStruct((E, T), jnp.float32),
    )(Wg, xf)
    xs, pos2, scl, tmaps = _route(scoresT, xf)
    ys = _ffn(tmaps, xs, scl, W1, W2, W3)
    out = _combine(ys, pos2)
    return out.reshape(B, T, DIM)
